# baseline (speedup 1.0000x reference)
import functools

import jax
import jax.numpy as jnp
from jax import lax
from jax.experimental import pallas as pl
from jax.experimental.pallas import tpu as pltpu
from jax.experimental.pallas import tpu_sc as plsc


def _xproj_body(t2, x_ref, wiou_ref, biou_ref, wf_ref, bf_ref, o_ref, nxf_ref):
    x = x_ref[...].astype(jnp.bfloat16)
    o_ref[...] = lax.dot_general(
        x, wiou_ref[...], (((1,), (1,)), ((), ())),
        preferred_element_type=jnp.float32) + biou_ref[...]
    nxf_ref[0] = -(lax.dot_general(
        x[:t2, :], wf_ref[...], (((1,), (1,)), ((), ())),
        preferred_element_type=jnp.float32) + bf_ref[...])


def _x_projection(x2d, w_iou, b_iou2, w_f, b_f2, t2):
    n, d = x2d.shape
    th = w_iou.shape[0]
    h = w_f.shape[0]
    blk = 512
    return pl.pallas_call(
        functools.partial(_xproj_body, t2),
        grid=(n // blk,),
        in_specs=[pl.BlockSpec((blk, d), lambda i: (i, 0)),
                  pl.BlockSpec((th, d), lambda i: (0, 0)),
                  pl.BlockSpec((1, th), lambda i: (0, 0)),
                  pl.BlockSpec((h, d), lambda i: (0, 0)),
                  pl.BlockSpec((1, h), lambda i: (0, 0))],
        out_specs=(pl.BlockSpec((blk, th), lambda i: (i, 0)),
                   pl.BlockSpec((1, t2, h), lambda i: (i, 0, 0))),
        out_shape=(jax.ShapeDtypeStruct((n, th), jnp.float32),
                   jax.ShapeDtypeStruct((n // blk, t2, h), jnp.float32)),
    )(x2d, w_iou, b_iou2, w_f, b_f2)


def _make_gates(first, last, n, h):
    blk = 2048

    def body(*refs):
        if first:
            x_ref, biou_ref, whf_ref, bhf_ref = refs[:4]
            outs = refs[4:]
        elif last:
            x_ref, hj_ref, cr_ref, wiou_ref, biou_ref = refs[:5]
            outs = refs[5:]
        else:
            x_ref, hj_ref, cr_ref, wiou_ref, biou_ref, whf_ref, bhf_ref = refs[:7]
            outs = refs[7:]
        x = x_ref[...]
        x_i = x[:, 0 * h:1 * h]
        x_o = x[:, 1 * h:2 * h]
        x_u = x[:, 2 * h:3 * h]
        if first:
            hio = biou_ref[...]
        else:
            hio = lax.dot_general(
                hj_ref[...].astype(jnp.bfloat16), wiou_ref[...],
                (((1,), (1,)), ((), ())),
                preferred_element_type=jnp.float32) + biou_ref[...]
        h_i = hio[:, 0 * h:1 * h]
        h_o = hio[:, 1 * h:2 * h]
        h_u = hio[:, 2 * h:3 * h]
        i_g = jax.nn.sigmoid(x_i + h_i)
        o_g = jax.nn.sigmoid(x_o + h_o)
        u_g = jnp.tanh(x_u + h_u)
        c = i_g * u_g
        if not first:
            c = c + cr_ref[...]
        hh = o_g * jnp.tanh(c)
        if last:
            outs[0][...] = hh
        else:
            nnz_h = jnp.sum((hh != 0).astype(jnp.float32), axis=1, keepdims=True)
            sc_h = jnp.where(nnz_h > 2.0, 2.0 / (nnz_h + 1e-7), 1.0)
            nnz_c = jnp.sum((c != 0).astype(jnp.float32), axis=1, keepdims=True)
            sc_c = jnp.where(nnz_c > 2.0, 2.0 / (nnz_c + 1e-7), 1.0)
            sh = hh * sc_h
            scl = c * sc_c
            outs[0][...] = sh
            outs[1][...] = scl
            outs[2][...] = lax.dot_general(
                sh.astype(jnp.bfloat16), whf_ref[...], (((1,), (1,)), ((), ())),
                preferred_element_type=jnp.float32) + bhf_ref[...]

    full = lambda shape: pl.BlockSpec(shape, lambda i: (0, 0))
    row_blk = lambda cols: pl.BlockSpec((blk, cols), lambda i: (i, 0))

    if first:
        in_specs = [row_blk(3 * h), full((1, 3 * h)), full((h, h)), full((1, h))]
    elif last:
        in_specs = [row_blk(3 * h), row_blk(h), row_blk(h),
                    full((3 * h, h)), full((1, 3 * h))]
    else:
        in_specs = [row_blk(3 * h), row_blk(h), row_blk(h),
                    full((3 * h, h)), full((1, 3 * h)), full((h, h)), full((1, h))]
    if last:
        out_specs = row_blk(h)
        out_shape = jax.ShapeDtypeStruct((n, h), jnp.float32)
    else:
        out_specs = (row_blk(h), row_blk(h), row_blk(h))
        out_shape = tuple(jax.ShapeDtypeStruct((n, h), jnp.float32) for _ in range(3))
    return pl.pallas_call(
        body,
        grid=(n // blk,),
        in_specs=in_specs,
        out_specs=out_specs,
        out_shape=out_shape,
    )


def _make_sc_childsum(n_nodes, t2, h, t1):
    nc, ns = 2, 16
    nw = nc * ns
    npt = n_nodes // nw
    ch_nodes = 16
    nch = npt // ch_nodes
    idxc = ch_nodes * t2
    nv = h // 16
    mesh = plsc.VectorSubcoreMesh(core_axis_name="c", subcore_axis_name="s")

    @functools.partial(
        pl.kernel,
        mesh=mesh,
        out_type=[jax.ShapeDtypeStruct((n_nodes, h), jnp.float32),
                  jax.ShapeDtypeStruct((n_nodes, h), jnp.float32)],
        scratch_types=[
            pltpu.VMEM((npt * t2,), jnp.int32),
            pltpu.VMEM((npt * t2 * 16,), jnp.float32),
            pltpu.VMEM((t2, h), jnp.float32),
            pltpu.VMEM((idxc, h), jnp.float32),
            pltpu.VMEM((idxc, h), jnp.float32),
            pltpu.VMEM((idxc, h), jnp.float32),
            pltpu.VMEM((idxc, h), jnp.float32),
            pltpu.VMEM((idxc, h), jnp.float32),
            pltpu.VMEM((idxc, h), jnp.float32),
            pltpu.VMEM((ch_nodes, h), jnp.float32),
            pltpu.VMEM((ch_nodes, h), jnp.float32),
            pltpu.SemaphoreType.DMA,
            pltpu.SemaphoreType.DMA,
        ],
    )
    def sc_kernel(sh_hbm, sc_hbm, hf_hbm, idx_hbm, msk_hbm, nxf_hbm,
                  hj_hbm, cr_hbm,
                  idx_v, msk_v, nxf_v, gh_a, gc_a, gf_a, gh_b, gc_b, gf_b,
                  hj_v, cr_v, sem_a, sem_b):
        wid = lax.axis_index("s") * nc + lax.axis_index("c")
        base = wid * npt
        b = base // t1
        pltpu.sync_copy(idx_hbm.at[pl.ds(base * t2, npt * t2)], idx_v)
        pltpu.sync_copy(msk_hbm.at[pl.ds(base * t2 * 16, npt * t2 * 16)], msk_v)
        pltpu.sync_copy(nxf_hbm.at[pl.ds(b * t2, t2)], nxf_v)

        def fire(ci, gh, gc, gf, sem):
            sl = idx_v.at[pl.ds(ci * idxc, idxc)]
            pltpu.async_copy(sh_hbm.at[sl], gh, sem)
            pltpu.async_copy(sc_hbm.at[sl], gc, sem)
            pltpu.async_copy(hf_hbm.at[sl], gf, sem)

        def drain(gh, gc, gf, sem):
            d = idx_v.at[pl.ds(0, idxc)]
            pltpu.make_async_copy(sh_hbm.at[d], gh, sem).wait()
            pltpu.make_async_copy(sc_hbm.at[d], gc, sem).wait()
            pltpu.make_async_copy(hf_hbm.at[d], gf, sem).wait()

        def compute(ci, gh, gc, gf):
            ioff = ci * idxc

            vg = 2

            def node_pl(i2):
                ni = i2 // (nv // vg)
                g = i2 % (nv // vg)
                rbase = ni * t2
                m_vecs = [msk_v[pl.ds((ioff + rbase + k) * 16, 16)]
                          for k in range(t2)]
                voff = g * vg * 16
                for j in range(vg):
                    sl = pl.ds(voff + 16 * j, 16)
                    acc_h = jnp.zeros((16,), jnp.float32)
                    acc_c = jnp.zeros((16,), jnp.float32)
                    for k in range(t2):
                        rk = rbase + k
                        acc_h = acc_h + m_vecs[k] * gh[rk, sl]
                        d = jnp.exp(nxf_v[k, sl] - gf[rk, sl]) + 1.0
                        acc_c = acc_c + (m_vecs[k] * gc[rk, sl]) / d
                    hj_v[ni, sl] = acc_h
                    cr_v[ni, sl] = acc_c

            plsc.parallel_loop(0, ch_nodes * (nv // vg), unroll=8)(node_pl)
            orow = base + ci * ch_nodes
            pltpu.sync_copy(hj_v, hj_hbm.at[pl.ds(orow, ch_nodes)])
            pltpu.sync_copy(cr_v, cr_hbm.at[pl.ds(orow, ch_nodes)])

        fire(0, gh_a, gc_a, gf_a, sem_a)

        def pair(i, carry):
            ci = 2 * i

            @pl.when(ci + 1 < nch)
            def _():
                fire(ci + 1, gh_b, gc_b, gf_b, sem_b)

            drain(gh_a, gc_a, gf_a, sem_a)
            compute(ci, gh_a, gc_a, gf_a)

            @pl.when(ci + 2 < nch)
            def _():
                fire(ci + 2, gh_a, gc_a, gf_a, sem_a)

            drain(gh_b, gc_b, gf_b, sem_b)
            compute(ci + 1, gh_b, gc_b, gf_b)
            return carry

        lax.fori_loop(0, nch // 2, pair, 0)

    return sc_kernel


def kernel(token_encodings, trees, child_mask, max_depth,
           W_x_iouf, b_x_iouf, W_h_iou, b_h_iou, W_h_f, b_h_f):
    bsz, t1, in_dim = token_encodings.shape
    t2 = trees.shape[2]
    h = W_h_f.shape[0]
    n = bsz * t1
    rows = t1 + 2

    x2d = token_encodings.reshape(n, in_dim)
    w_x_iou = W_x_iouf[:3 * h].astype(jnp.bfloat16)
    w_x_f = W_x_iouf[3 * h:].astype(jnp.bfloat16)
    b_x_iou2 = b_x_iouf[:3 * h].reshape(1, -1)
    b_x_f2 = b_x_iouf[3 * h:].reshape(1, -1)
    x_iou, nxf3 = _x_projection(x2d, w_x_iou, b_x_iou2, w_x_f, b_x_f2, t2)
    nxf = nxf3.reshape(bsz * t2, h)

    tr = trees.reshape(-1).astype(jnp.int32)
    r = tr % rows
    pad = r < 2
    idx_eff = jnp.where(pad, 0, (tr // rows) * t1 + (r - 2)).astype(jnp.int32)
    msk_eff = jnp.where(pad, 0.0, child_mask.reshape(-1))
    msk16 = jnp.repeat(msk_eff, 16)

    biou2 = b_h_iou.reshape(1, -1)
    bhf2 = b_h_f.reshape(1, -1)
    w_h_iou = W_h_iou.astype(jnp.bfloat16)
    w_h_f = W_h_f.astype(jnp.bfloat16)

    gates_first = _make_gates(True, False, n, h)
    gates_mid = _make_gates(False, False, n, h)
    gates_last = _make_gates(False, True, n, h)
    sc_childsum = _make_sc_childsum(n, t2, h, t1)

    sh, scl, hf = gates_first(x_iou, biou2, w_h_f, bhf2)

    def mid(_, carry):
        sh, scl, hf = carry
        hj, cr = sc_childsum(sh, scl, hf, idx_eff, msk16, nxf)
        return gates_mid(x_iou, hj, cr, w_h_iou, biou2, w_h_f, bhf2)

    sh, scl, hf = lax.fori_loop(1, max_depth - 1, mid, (sh, scl, hf))
    hj, cr = sc_childsum(sh, scl, hf, idx_eff, msk16, nxf)
    out = gates_last(x_iou, hj, cr, w_h_iou, biou2)
    return out.reshape(bsz, t1, h)


# speedup vs baseline: 1.8726x; 1.8726x over previous
import functools

import jax
import jax.numpy as jnp
from jax import lax
from jax.experimental import pallas as pl
from jax.experimental.pallas import tpu as pltpu
from jax.experimental.pallas import tpu_sc as plsc


def _xproj_body(t2, x_ref, wiou_ref, biou_ref, wf_ref, bf_ref, o_ref, nxf_ref):
    x = x_ref[...].astype(jnp.bfloat16)
    o_ref[...] = (lax.dot_general(
        x, wiou_ref[...], (((1,), (1,)), ((), ())),
        preferred_element_type=jnp.float32) + biou_ref[...]).astype(jnp.bfloat16)
    nxf_ref[0] = -(lax.dot_general(
        x[:t2, :], wf_ref[...], (((1,), (1,)), ((), ())),
        preferred_element_type=jnp.float32) + bf_ref[...])


def _x_projection(x2d, w_iou, b_iou2, w_f, b_f2, t2):
    n, d = x2d.shape
    th = w_iou.shape[0]
    h = w_f.shape[0]
    blk = 512
    return pl.pallas_call(
        functools.partial(_xproj_body, t2),
        grid=(n // blk,),
        in_specs=[pl.BlockSpec((blk, d), lambda i: (i, 0)),
                  pl.BlockSpec((th, d), lambda i: (0, 0)),
                  pl.BlockSpec((1, th), lambda i: (0, 0)),
                  pl.BlockSpec((h, d), lambda i: (0, 0)),
                  pl.BlockSpec((1, h), lambda i: (0, 0))],
        out_specs=(pl.BlockSpec((blk, th), lambda i: (i, 0)),
                   pl.BlockSpec((1, t2, h), lambda i: (i, 0, 0))),
        out_shape=(jax.ShapeDtypeStruct((n, th), jnp.bfloat16),
                   jax.ShapeDtypeStruct((n // blk, t2, h), jnp.float32)),
    )(x2d, w_iou, b_iou2, w_f, b_f2)


def _make_gates(first, last, n, h):
    blk = 2048

    def body(*refs):
        if first:
            x_ref, biou_ref, whf_ref, bhf_ref = refs[:4]
            outs = refs[4:]
        elif last:
            x_ref, hj_ref, cr_ref, wiou_ref, biou_ref = refs[:5]
            outs = refs[5:]
        else:
            x_ref, hj_ref, cr_ref, wiou_ref, biou_ref, whf_ref, bhf_ref = refs[:7]
            outs = refs[7:]
        x = x_ref[...].astype(jnp.float32)
        x_i = x[:, 0 * h:1 * h]
        x_o = x[:, 1 * h:2 * h]
        x_u = x[:, 2 * h:3 * h]
        if first:
            hio = biou_ref[...]
        else:
            hio = lax.dot_general(
                hj_ref[...].astype(jnp.bfloat16), wiou_ref[...],
                (((1,), (1,)), ((), ())),
                preferred_element_type=jnp.float32) + biou_ref[...]
        h_i = hio[:, 0 * h:1 * h]
        h_o = hio[:, 1 * h:2 * h]
        h_u = hio[:, 2 * h:3 * h]
        i_g = jax.nn.sigmoid(x_i + h_i)
        o_g = jax.nn.sigmoid(x_o + h_o)
        u_g = jnp.tanh(x_u + h_u)
        c = i_g * u_g
        if not first:
            c = c + cr_ref[...]
        hh = o_g * jnp.tanh(c)
        if last:
            outs[0][...] = hh
        else:
            nnz_h = jnp.sum((hh != 0).astype(jnp.float32), axis=1, keepdims=True)
            sc_h = jnp.where(nnz_h > 2.0, 2.0 / (nnz_h + 1e-7), 1.0)
            nnz_c = jnp.sum((c != 0).astype(jnp.float32), axis=1, keepdims=True)
            sc_c = jnp.where(nnz_c > 2.0, 2.0 / (nnz_c + 1e-7), 1.0)
            sh = hh * sc_h
            scl = c * sc_c
            outs[0][...] = sh
            outs[1][...] = scl
            outs[2][...] = lax.dot_general(
                sh.astype(jnp.bfloat16), whf_ref[...], (((1,), (1,)), ((), ())),
                preferred_element_type=jnp.float32) + bhf_ref[...]

    full = lambda shape: pl.BlockSpec(shape, lambda i: (0, 0))
    row_blk = lambda cols: pl.BlockSpec((blk, cols), lambda i: (i, 0))

    if first:
        in_specs = [row_blk(3 * h), full((1, 3 * h)), full((h, h)), full((1, h))]
    elif last:
        in_specs = [row_blk(3 * h), row_blk(h), row_blk(h),
                    full((3 * h, h)), full((1, 3 * h))]
    else:
        in_specs = [row_blk(3 * h), row_blk(h), row_blk(h),
                    full((3 * h, h)), full((1, 3 * h)), full((h, h)), full((1, h))]
    if last:
        out_specs = row_blk(h)
        out_shape = jax.ShapeDtypeStruct((n, h), jnp.float32)
    else:
        out_specs = (row_blk(h), row_blk(h), row_blk(h))
        out_shape = tuple(jax.ShapeDtypeStruct((n, h), jnp.float32) for _ in range(3))
    return pl.pallas_call(
        body,
        grid=(n // blk,),
        in_specs=in_specs,
        out_specs=out_specs,
        out_shape=out_shape,
    )


def _make_sc_childsum(n_nodes, t2, h, t1):
    nc, ns = 2, 16
    nw = nc * ns
    npt = n_nodes // nw
    ch_nodes = 16
    nch = npt // ch_nodes
    idxc = ch_nodes * t2
    nv = h // 16
    mesh = plsc.VectorSubcoreMesh(core_axis_name="c", subcore_axis_name="s")

    @functools.partial(
        pl.kernel,
        mesh=mesh,
        out_type=[jax.ShapeDtypeStruct((n_nodes, h), jnp.float32),
                  jax.ShapeDtypeStruct((n_nodes, h), jnp.float32)],
        scratch_types=[
            pltpu.VMEM((npt * t2,), jnp.int32),
            pltpu.VMEM((npt * t2 * 16,), jnp.float32),
            pltpu.VMEM((t2, h), jnp.float32),
            pltpu.VMEM((idxc, h), jnp.float32),
            pltpu.VMEM((idxc, h), jnp.float32),
            pltpu.VMEM((idxc, h), jnp.float32),
            pltpu.VMEM((idxc, h), jnp.float32),
            pltpu.VMEM((idxc, h), jnp.float32),
            pltpu.VMEM((idxc, h), jnp.float32),
            pltpu.VMEM((ch_nodes, h), jnp.float32),
            pltpu.VMEM((ch_nodes, h), jnp.float32),
            pltpu.SemaphoreType.DMA,
            pltpu.SemaphoreType.DMA,
        ],
    )
    def sc_kernel(sh_hbm, sc_hbm, hf_hbm, idx_hbm, msk_hbm, nxf_hbm,
                  hj_hbm, cr_hbm,
                  idx_v, msk_v, nxf_v, gh_a, gc_a, gf_a, gh_b, gc_b, gf_b,
                  hj_v, cr_v, sem_a, sem_b):
        wid = lax.axis_index("s") * nc + lax.axis_index("c")
        base = wid * npt
        b = base // t1
        pltpu.sync_copy(idx_hbm.at[pl.ds(base * t2, npt * t2)], idx_v)
        pltpu.sync_copy(msk_hbm.at[pl.ds(base * t2 * 16, npt * t2 * 16)], msk_v)
        pltpu.sync_copy(nxf_hbm.at[pl.ds(b * t2, t2)], nxf_v)

        def fire(ci, gh, gc, gf, sem):
            sl = idx_v.at[pl.ds(ci * idxc, idxc)]
            pltpu.async_copy(sh_hbm.at[sl], gh, sem)
            pltpu.async_copy(sc_hbm.at[sl], gc, sem)
            pltpu.async_copy(hf_hbm.at[sl], gf, sem)

        def drain(gh, gc, gf, sem):
            d = idx_v.at[pl.ds(0, idxc)]
            pltpu.make_async_copy(sh_hbm.at[d], gh, sem).wait()
            pltpu.make_async_copy(sc_hbm.at[d], gc, sem).wait()
            pltpu.make_async_copy(hf_hbm.at[d], gf, sem).wait()

        def compute(ci, gh, gc, gf):
            ioff = ci * idxc

            vg = 2

            def node_pl(i2):
                ni = i2 // (nv // vg)
                g = i2 % (nv // vg)
                rbase = ni * t2
                m_vecs = [msk_v[pl.ds((ioff + rbase + k) * 16, 16)]
                          for k in range(t2)]
                voff = g * vg * 16
                for j in range(vg):
                    sl = pl.ds(voff + 16 * j, 16)
                    acc_h = jnp.zeros((16,), jnp.float32)
                    acc_c = jnp.zeros((16,), jnp.float32)
                    for k in range(t2):
                        rk = rbase + k
                        acc_h = acc_h + m_vecs[k] * gh[rk, sl]
                        d = jnp.exp(nxf_v[k, sl] - gf[rk, sl]) + 1.0
                        acc_c = acc_c + (m_vecs[k] * gc[rk, sl]) / d
                    hj_v[ni, sl] = acc_h
                    cr_v[ni, sl] = acc_c

            plsc.parallel_loop(0, ch_nodes * (nv // vg), unroll=4)(node_pl)
            orow = base + ci * ch_nodes
            pltpu.sync_copy(hj_v, hj_hbm.at[pl.ds(orow, ch_nodes)])
            pltpu.sync_copy(cr_v, cr_hbm.at[pl.ds(orow, ch_nodes)])

        fire(0, gh_a, gc_a, gf_a, sem_a)

        def pair(i, carry):
            ci = 2 * i

            @pl.when(ci + 1 < nch)
            def _():
                fire(ci + 1, gh_b, gc_b, gf_b, sem_b)

            drain(gh_a, gc_a, gf_a, sem_a)
            compute(ci, gh_a, gc_a, gf_a)

            @pl.when(ci + 2 < nch)
            def _():
                fire(ci + 2, gh_a, gc_a, gf_a, sem_a)

            drain(gh_b, gc_b, gf_b, sem_b)
            compute(ci + 1, gh_b, gc_b, gf_b)
            return carry

        lax.fori_loop(0, nch // 2, pair, 0)

    return sc_kernel


def kernel(token_encodings, trees, child_mask, max_depth,
           W_x_iouf, b_x_iouf, W_h_iou, b_h_iou, W_h_f, b_h_f):
    bsz, t1, in_dim = token_encodings.shape
    t2 = trees.shape[2]
    h = W_h_f.shape[0]
    n = bsz * t1
    rows = t1 + 2

    x2d = token_encodings.reshape(n, in_dim)
    w_x_iou = W_x_iouf[:3 * h].astype(jnp.bfloat16)
    w_x_f = W_x_iouf[3 * h:].astype(jnp.bfloat16)
    b_x_iou2 = b_x_iouf[:3 * h].reshape(1, -1)
    b_x_f2 = b_x_iouf[3 * h:].reshape(1, -1)
    x_iou, nxf3 = _x_projection(x2d, w_x_iou, b_x_iou2, w_x_f, b_x_f2, t2)
    nxf = nxf3.reshape(bsz * t2, h)

    tr = trees.reshape(-1).astype(jnp.int32)
    r = tr % rows
    pad = r < 2
    idx_eff = jnp.where(pad, 0, (tr // rows) * t1 + (r - 2)).astype(jnp.int32)
    msk_eff = jnp.where(pad, 0.0, child_mask.reshape(-1))
    msk16 = jnp.repeat(msk_eff, 16)

    biou2 = b_h_iou.reshape(1, -1)
    bhf2 = b_h_f.reshape(1, -1)
    w_h_iou = W_h_iou.astype(jnp.bfloat16)
    w_h_f = W_h_f.astype(jnp.bfloat16)

    gates_first = _make_gates(True, False, n, h)
    gates_mid = _make_gates(False, False, n, h)
    gates_last = _make_gates(False, True, n, h)
    sc_childsum = _make_sc_childsum(n, t2, h, t1)

    sh, scl, hf = gates_first(x_iou, biou2, w_h_f, bhf2)

    def mid(_, carry):
        sh, scl, hf = carry
        hj, cr = sc_childsum(sh, scl, hf, idx_eff, msk16, nxf)
        return gates_mid(x_iou, hj, cr, w_h_iou, biou2, w_h_f, bhf2)

    sh, scl, hf = lax.fori_loop(1, max_depth - 1, mid, (sh, scl, hf))
    hj, cr = sc_childsum(sh, scl, hf, idx_eff, msk16, nxf)
    out = gates_last(x_iou, hj, cr, w_h_iou, biou2)
    return out.reshape(bsz, t1, h)


# speedup vs baseline: 1.9164x; 1.0234x over previous
import functools

import jax
import jax.numpy as jnp
from jax import lax
from jax.experimental import pallas as pl
from jax.experimental.pallas import tpu as pltpu
from jax.experimental.pallas import tpu_sc as plsc


def _xproj_body(t2, x_ref, wiou_ref, biou_ref, wf_ref, bf_ref, o_ref, nxf_ref):
    x = x_ref[...].astype(jnp.bfloat16)
    o_ref[...] = (lax.dot_general(
        x, wiou_ref[...], (((1,), (1,)), ((), ())),
        preferred_element_type=jnp.float32) + biou_ref[...]).astype(jnp.bfloat16)
    nxf_ref[0] = -(lax.dot_general(
        x[:t2, :], wf_ref[...], (((1,), (1,)), ((), ())),
        preferred_element_type=jnp.float32) + bf_ref[...])


def _x_projection(x2d, w_iou, b_iou2, w_f, b_f2, t2):
    n, d = x2d.shape
    th = w_iou.shape[0]
    h = w_f.shape[0]
    blk = 512
    return pl.pallas_call(
        functools.partial(_xproj_body, t2),
        grid=(n // blk,),
        in_specs=[pl.BlockSpec((blk, d), lambda i: (i, 0)),
                  pl.BlockSpec((th, d), lambda i: (0, 0)),
                  pl.BlockSpec((1, th), lambda i: (0, 0)),
                  pl.BlockSpec((h, d), lambda i: (0, 0)),
                  pl.BlockSpec((1, h), lambda i: (0, 0))],
        out_specs=(pl.BlockSpec((blk, th), lambda i: (i, 0)),
                   pl.BlockSpec((1, t2, h), lambda i: (i, 0, 0))),
        out_shape=(jax.ShapeDtypeStruct((n, th), jnp.bfloat16),
                   jax.ShapeDtypeStruct((n // blk, t2, h), jnp.float32)),
    )(x2d, w_iou, b_iou2, w_f, b_f2)


def _make_gates(first, last, n, h, t1=512, t2=4):
    blk = 2048
    bpb = blk // t1

    def body(*refs):
        if first:
            (x_ref, wxiou_ref, bxiou_ref, wxf_ref, bxf_ref,
             biou_ref, whf_ref, bhf_ref) = refs[:8]
            outs = refs[8:]
        elif last:
            x_ref, hj_ref, cr_ref, wiou_ref, biou_ref = refs[:5]
            outs = refs[5:]
        else:
            x_ref, hj_ref, cr_ref, wiou_ref, biou_ref, whf_ref, bhf_ref = refs[:7]
            outs = refs[7:]
        if first:
            xt = x_ref[...].astype(jnp.bfloat16)
            x = lax.dot_general(
                xt, wxiou_ref[...], (((1,), (1,)), ((), ())),
                preferred_element_type=jnp.float32) + bxiou_ref[...]
            outs[3][...] = x.astype(jnp.bfloat16)
            xt4 = jnp.concatenate([xt[b * t1:b * t1 + t2] for b in range(bpb)],
                                  axis=0)
            outs[4][0] = -(lax.dot_general(
                xt4, wxf_ref[...], (((1,), (1,)), ((), ())),
                preferred_element_type=jnp.float32) + bxf_ref[...])
        else:
            x = x_ref[...].astype(jnp.float32)
        x_i = x[:, 0 * h:1 * h]
        x_o = x[:, 1 * h:2 * h]
        x_u = x[:, 2 * h:3 * h]
        if first:
            hio = biou_ref[...]
        else:
            hio = lax.dot_general(
                hj_ref[...].astype(jnp.bfloat16), wiou_ref[...],
                (((1,), (1,)), ((), ())),
                preferred_element_type=jnp.float32) + biou_ref[...]
        h_i = hio[:, 0 * h:1 * h]
        h_o = hio[:, 1 * h:2 * h]
        h_u = hio[:, 2 * h:3 * h]
        i_g = jax.nn.sigmoid(x_i + h_i)
        o_g = jax.nn.sigmoid(x_o + h_o)
        u_g = jnp.tanh(x_u + h_u)
        c = i_g * u_g
        if not first:
            c = c + cr_ref[...]
        hh = o_g * jnp.tanh(c)
        if last:
            outs[0][...] = hh
        else:
            nnz_h = jnp.sum((hh != 0).astype(jnp.float32), axis=1, keepdims=True)
            sc_h = jnp.where(nnz_h > 2.0, 2.0 / (nnz_h + 1e-7), 1.0)
            nnz_c = jnp.sum((c != 0).astype(jnp.float32), axis=1, keepdims=True)
            sc_c = jnp.where(nnz_c > 2.0, 2.0 / (nnz_c + 1e-7), 1.0)
            sh = hh * sc_h
            scl = c * sc_c
            outs[0][...] = sh
            outs[1][...] = scl
            outs[2][...] = lax.dot_general(
                sh.astype(jnp.bfloat16), whf_ref[...], (((1,), (1,)), ((), ())),
                preferred_element_type=jnp.float32) + bhf_ref[...]

    full = lambda shape: pl.BlockSpec(shape, lambda i: (0, 0))
    row_blk = lambda cols: pl.BlockSpec((blk, cols), lambda i: (i, 0))

    if first:
        in_specs = [row_blk(h), full((3 * h, h)), full((1, 3 * h)),
                    full((h, h)), full((1, h)),
                    full((1, 3 * h)), full((h, h)), full((1, h))]
    elif last:
        in_specs = [row_blk(3 * h), row_blk(h), row_blk(h),
                    full((3 * h, h)), full((1, 3 * h))]
    else:
        in_specs = [row_blk(3 * h), row_blk(h), row_blk(h),
                    full((3 * h, h)), full((1, 3 * h)), full((h, h)), full((1, h))]
    if last:
        out_specs = row_blk(h)
        out_shape = jax.ShapeDtypeStruct((n, h), jnp.float32)
    elif first:
        out_specs = (row_blk(h), row_blk(h), row_blk(h), row_blk(3 * h),
                     pl.BlockSpec((1, bpb * t2, h), lambda i: (i, 0, 0)))
        out_shape = (jax.ShapeDtypeStruct((n, h), jnp.float32),
                     jax.ShapeDtypeStruct((n, h), jnp.float32),
                     jax.ShapeDtypeStruct((n, h), jnp.float32),
                     jax.ShapeDtypeStruct((n, 3 * h), jnp.bfloat16),
                     jax.ShapeDtypeStruct((n // blk, bpb * t2, h), jnp.float32))
    else:
        out_specs = (row_blk(h), row_blk(h), row_blk(h))
        out_shape = tuple(jax.ShapeDtypeStruct((n, h), jnp.float32) for _ in range(3))
    return pl.pallas_call(
        body,
        grid=(n // blk,),
        in_specs=in_specs,
        out_specs=out_specs,
        out_shape=out_shape,
    )


def _make_sc_childsum(n_nodes, t2, h, t1):
    nc, ns = 2, 16
    nw = nc * ns
    npt = n_nodes // nw
    ch_nodes = 16
    nch = npt // ch_nodes
    idxc = ch_nodes * t2
    nv = h // 16
    mesh = plsc.VectorSubcoreMesh(core_axis_name="c", subcore_axis_name="s")

    @functools.partial(
        pl.kernel,
        mesh=mesh,
        out_type=[jax.ShapeDtypeStruct((n_nodes, h), jnp.float32),
                  jax.ShapeDtypeStruct((n_nodes, h), jnp.float32)],
        scratch_types=[
            pltpu.VMEM((npt * t2,), jnp.int32),
            pltpu.VMEM((npt * t2 * 16,), jnp.float32),
            pltpu.VMEM((t2, h), jnp.float32),
            pltpu.VMEM((idxc, h), jnp.float32),
            pltpu.VMEM((idxc, h), jnp.float32),
            pltpu.VMEM((idxc, h), jnp.float32),
            pltpu.VMEM((idxc, h), jnp.float32),
            pltpu.VMEM((idxc, h), jnp.float32),
            pltpu.VMEM((idxc, h), jnp.float32),
            pltpu.VMEM((ch_nodes, h), jnp.float32),
            pltpu.VMEM((ch_nodes, h), jnp.float32),
            pltpu.SemaphoreType.DMA,
            pltpu.SemaphoreType.DMA,
        ],
    )
    def sc_kernel(sh_hbm, sc_hbm, hf_hbm, idx_hbm, msk_hbm, nxf_hbm,
                  hj_hbm, cr_hbm,
                  idx_v, msk_v, nxf_v, gh_a, gc_a, gf_a, gh_b, gc_b, gf_b,
                  hj_v, cr_v, sem_a, sem_b):
        wid = lax.axis_index("s") * nc + lax.axis_index("c")
        base = wid * npt
        b = base // t1
        pltpu.sync_copy(idx_hbm.at[pl.ds(base * t2, npt * t2)], idx_v)
        pltpu.sync_copy(msk_hbm.at[pl.ds(base * t2 * 16, npt * t2 * 16)], msk_v)
        pltpu.sync_copy(nxf_hbm.at[pl.ds(b * t2, t2)], nxf_v)

        def fire(ci, gh, gc, gf, sem):
            sl = idx_v.at[pl.ds(ci * idxc, idxc)]
            pltpu.async_copy(sh_hbm.at[sl], gh, sem)
            pltpu.async_copy(sc_hbm.at[sl], gc, sem)
            pltpu.async_copy(hf_hbm.at[sl], gf, sem)

        def drain(gh, gc, gf, sem):
            d = idx_v.at[pl.ds(0, idxc)]
            pltpu.make_async_copy(sh_hbm.at[d], gh, sem).wait()
            pltpu.make_async_copy(sc_hbm.at[d], gc, sem).wait()
            pltpu.make_async_copy(hf_hbm.at[d], gf, sem).wait()

        def compute(ci, gh, gc, gf):
            ioff = ci * idxc

            vg = 2

            def node_pl(i2):
                ni = i2 // (nv // vg)
                g = i2 % (nv // vg)
                rbase = ni * t2
                m_vecs = [msk_v[pl.ds((ioff + rbase + k) * 16, 16)]
                          for k in range(t2)]
                voff = g * vg * 16
                for j in range(vg):
                    sl = pl.ds(voff + 16 * j, 16)
                    acc_h = jnp.zeros((16,), jnp.float32)
                    acc_c = jnp.zeros((16,), jnp.float32)
                    for k in range(t2):
                        rk = rbase + k
                        acc_h = acc_h + m_vecs[k] * gh[rk, sl]
                        d = jnp.exp(nxf_v[k, sl] - gf[rk, sl]) + 1.0
                        acc_c = acc_c + (m_vecs[k] * gc[rk, sl]) / d
                    hj_v[ni, sl] = acc_h
                    cr_v[ni, sl] = acc_c

            plsc.parallel_loop(0, ch_nodes * (nv // vg), unroll=4)(node_pl)
            orow = base + ci * ch_nodes
            pltpu.sync_copy(hj_v, hj_hbm.at[pl.ds(orow, ch_nodes)])
            pltpu.sync_copy(cr_v, cr_hbm.at[pl.ds(orow, ch_nodes)])

        fire(0, gh_a, gc_a, gf_a, sem_a)

        def pair(i, carry):
            ci = 2 * i

            @pl.when(ci + 1 < nch)
            def _():
                fire(ci + 1, gh_b, gc_b, gf_b, sem_b)

            drain(gh_a, gc_a, gf_a, sem_a)
            compute(ci, gh_a, gc_a, gf_a)

            @pl.when(ci + 2 < nch)
            def _():
                fire(ci + 2, gh_a, gc_a, gf_a, sem_a)

            drain(gh_b, gc_b, gf_b, sem_b)
            compute(ci + 1, gh_b, gc_b, gf_b)
            return carry

        lax.fori_loop(0, nch // 2, pair, 0)

    return sc_kernel


def kernel(token_encodings, trees, child_mask, max_depth,
           W_x_iouf, b_x_iouf, W_h_iou, b_h_iou, W_h_f, b_h_f):
    bsz, t1, in_dim = token_encodings.shape
    t2 = trees.shape[2]
    h = W_h_f.shape[0]
    n = bsz * t1
    rows = t1 + 2

    x2d = token_encodings.reshape(n, in_dim)
    w_x_iou = W_x_iouf[:3 * h].astype(jnp.bfloat16)
    w_x_f = W_x_iouf[3 * h:].astype(jnp.bfloat16)
    b_x_iou2 = b_x_iouf[:3 * h].reshape(1, -1)
    b_x_f2 = b_x_iouf[3 * h:].reshape(1, -1)
    tr = trees.reshape(-1).astype(jnp.int32)
    r = tr % rows
    pad = r < 2
    idx_eff = jnp.where(pad, 0, (tr // rows) * t1 + (r - 2)).astype(jnp.int32)
    msk_eff = jnp.where(pad, 0.0, child_mask.reshape(-1))
    msk16 = jnp.repeat(msk_eff, 16)

    biou2 = b_h_iou.reshape(1, -1)
    bhf2 = b_h_f.reshape(1, -1)
    w_h_iou = W_h_iou.astype(jnp.bfloat16)
    w_h_f = W_h_f.astype(jnp.bfloat16)

    gates_first = _make_gates(True, False, n, h, t1, t2)
    gates_mid = _make_gates(False, False, n, h, t1, t2)
    gates_last = _make_gates(False, True, n, h, t1, t2)
    sc_childsum = _make_sc_childsum(n, t2, h, t1)

    sh, scl, hf, x_iou, nxf3 = gates_first(
        x2d, w_x_iou, b_x_iou2, w_x_f, b_x_f2, biou2, w_h_f, bhf2)
    nxf = nxf3.reshape(bsz * t2, h)

    def mid(_, carry):
        sh, scl, hf = carry
        hj, cr = sc_childsum(sh, scl, hf, idx_eff, msk16, nxf)
        return gates_mid(x_iou, hj, cr, w_h_iou, biou2, w_h_f, bhf2)

    sh, scl, hf = lax.fori_loop(1, max_depth - 1, mid, (sh, scl, hf))
    hj, cr = sc_childsum(sh, scl, hf, idx_eff, msk16, nxf)
    out = gates_last(x_iou, hj, cr, w_h_iou, biou2)
    return out.reshape(bsz, t1, h)
